# baseline (speedup 1.0000x reference)
import functools

import jax
import jax.numpy as jnp
from jax.experimental import pallas as pl
from jax.experimental.pallas import tpu as pltpu

_TILE = 8192
_NEG_BIG = -1e10
_NEG_PAD = -1e30


def _body(n_tiles, vocab, tile, x_ref, a_ref, w_ref, b_ref, o_ref,
          buf_ref, m_ref, s_ref):
    i = pl.program_id(0)

    @pl.when(i == 0)
    def _init():
        m_ref[...] = jnp.full_like(m_ref, _NEG_PAD)
        s_ref[...] = jnp.zeros_like(s_ref)

    @pl.when(i < n_tiles)
    def _phase1():
        logits = jax.lax.dot_general(
            x_ref[...], w_ref[...],
            (((1,), (1,)), ((), ())),
            preferred_element_type=jnp.float32)
        logits = logits + b_ref[...]
        masked = jnp.where(a_ref[...] == 0, jnp.float32(_NEG_BIG), logits)
        col = jax.lax.broadcasted_iota(jnp.int32, masked.shape, 1) + i * tile
        masked = jnp.where(col < vocab, masked, jnp.float32(_NEG_PAD))
        m_old = m_ref[:, 0:1]
        s_old = s_ref[:, 0:1]
        m_tile = jnp.max(masked, axis=1, keepdims=True)
        m_new = jnp.maximum(m_old, m_tile)
        s_new = (s_old * jnp.exp(m_old - m_new)
                 + jnp.sum(jnp.exp(masked - m_new), axis=1, keepdims=True))
        m_ref[:, 0:1] = m_new
        s_ref[:, 0:1] = s_new
        buf_ref[:, pl.ds(i * tile, tile)] = masked

    @pl.when(i >= n_tiles)
    def _phase2():
        j = i - n_tiles
        lse = m_ref[:, 0:1] + jnp.log(s_ref[:, 0:1])
        o_ref[...] = buf_ref[:, pl.ds(j * tile, tile)] - lse


def _build_call(batch, feat, vocab, tile):
    n = pl.cdiv(vocab, tile)
    body = functools.partial(_body, n, vocab, tile)
    grid = (2 * n,)
    in_specs = [
        pl.BlockSpec((batch, feat), lambda i: (0, 0)),
        pl.BlockSpec((batch, tile), lambda i: (0, jnp.minimum(i, n - 1))),
        pl.BlockSpec((tile, feat), lambda i: (jnp.minimum(i, n - 1), 0)),
        pl.BlockSpec((1, tile), lambda i: (0, jnp.minimum(i, n - 1))),
    ]
    out_spec = pl.BlockSpec((batch, tile), lambda i: (0, jnp.maximum(i - n, 0)))
    scratch = [
        pltpu.VMEM((batch, n * tile), jnp.float32),
        pltpu.VMEM((batch, 128), jnp.float32),
        pltpu.VMEM((batch, 128), jnp.float32),
    ]
    return pl.pallas_call(
        body,
        grid=grid,
        in_specs=in_specs,
        out_specs=out_spec,
        out_shape=jax.ShapeDtypeStruct((batch, vocab), jnp.float32),
        scratch_shapes=scratch,
        compiler_params=pltpu.CompilerParams(
            vmem_limit_bytes=100 * 1024 * 1024),
    )


def kernel(x, available_actions, W, b):
    batch, feat = x.shape
    vocab = W.shape[0]
    if available_actions.ndim == 1:
        available_actions = available_actions[None, :]
    available_actions = jnp.broadcast_to(available_actions, (batch, vocab))
    b2 = b.reshape(1, vocab)
    call = _build_call(batch, feat, vocab, _TILE)
    return call(x, available_actions, W, b2)


# speedup vs baseline: 1.0010x; 1.0010x over previous
import functools

import jax
import jax.numpy as jnp
from jax.experimental import pallas as pl
from jax.experimental.pallas import tpu as pltpu

_TILE = 8192
_NEG_BIG = -1e10
_NEG_PAD = -1e30


def _body(n_tiles, vocab, tile, x_ref, a_ref, w_ref, b_ref, o_ref,
          buf_ref, m_ref, s_ref):
    i = pl.program_id(0)

    @pl.when(i == 0)
    def _init():
        m_ref[...] = jnp.full_like(m_ref, _NEG_PAD)
        s_ref[...] = jnp.zeros_like(s_ref)

    @pl.when(i < n_tiles)
    def _phase1():
        logits = jax.lax.dot_general(
            x_ref[...], w_ref[...],
            (((1,), (1,)), ((), ())),
            preferred_element_type=jnp.float32)
        logits = logits + b_ref[...]
        masked = jnp.where(a_ref[...] == 0, jnp.float32(_NEG_BIG), logits)
        col = jax.lax.broadcasted_iota(jnp.int32, masked.shape, 1) + i * tile
        masked = jnp.where(col < vocab, masked, jnp.float32(_NEG_PAD))
        m_old = m_ref[:, 0:1]
        s_old = s_ref[:, 0:1]
        m_tile = jnp.max(masked, axis=1, keepdims=True)
        m_new = jnp.maximum(m_old, m_tile)
        s_new = (s_old * jnp.exp(m_old - m_new)
                 + jnp.sum(jnp.exp(masked - m_new), axis=1, keepdims=True))
        m_ref[:, 0:1] = m_new
        s_ref[:, 0:1] = s_new
        buf_ref[i] = masked

    @pl.when(i >= n_tiles)
    def _phase2():
        j = i - n_tiles
        lse = m_ref[:, 0:1] + jnp.log(s_ref[:, 0:1])
        o_ref[...] = buf_ref[j] - lse


def _build_call(batch, feat, vocab, tile):
    n = pl.cdiv(vocab, tile)
    body = functools.partial(_body, n, vocab, tile)
    grid = (2 * n,)
    in_specs = [
        pl.BlockSpec((batch, feat), lambda i: (0, 0)),
        pl.BlockSpec((batch, tile), lambda i: (0, jnp.minimum(i, n - 1))),
        pl.BlockSpec((tile, feat), lambda i: (jnp.minimum(i, n - 1), 0)),
        pl.BlockSpec((1, tile), lambda i: (0, jnp.minimum(i, n - 1))),
    ]
    out_spec = pl.BlockSpec((batch, tile), lambda i: (0, jnp.maximum(i - n, 0)))
    scratch = [
        pltpu.VMEM((n, batch, tile), jnp.float32),
        pltpu.VMEM((batch, 128), jnp.float32),
        pltpu.VMEM((batch, 128), jnp.float32),
    ]
    return pl.pallas_call(
        body,
        grid=grid,
        in_specs=in_specs,
        out_specs=out_spec,
        out_shape=jax.ShapeDtypeStruct((batch, vocab), jnp.float32),
        scratch_shapes=scratch,
        compiler_params=pltpu.CompilerParams(
            vmem_limit_bytes=100 * 1024 * 1024),
    )


def kernel(x, available_actions, W, b):
    batch, feat = x.shape
    vocab = W.shape[0]
    if available_actions.ndim == 1:
        available_actions = available_actions[None, :]
    available_actions = jnp.broadcast_to(available_actions, (batch, vocab))
    b2 = b.reshape(1, vocab)
    call = _build_call(batch, feat, vocab, _TILE)
    return call(x, available_actions, W, b2)


# speedup vs baseline: 1.1108x; 1.1097x over previous
import functools

import jax
import jax.numpy as jnp
from jax.experimental import pallas as pl
from jax.experimental.pallas import tpu as pltpu

_TILE = 16384
_NEG_BIG = -1e10
_NEG_PAD = -1e30


def _body(n_tiles, vocab, tile, x_ref, a_ref, w_ref, b_ref, o_ref,
          buf_ref, m_ref, s_ref):
    i = pl.program_id(0)

    @pl.when(i == 0)
    def _init():
        m_ref[...] = jnp.full_like(m_ref, _NEG_PAD)
        s_ref[...] = jnp.zeros_like(s_ref)

    @pl.when(i < n_tiles)
    def _phase1():
        logits = jax.lax.dot_general(
            x_ref[...].astype(jnp.bfloat16), w_ref[...].astype(jnp.bfloat16),
            (((1,), (1,)), ((), ())),
            preferred_element_type=jnp.float32)
        logits = logits + b_ref[...]
        masked = jnp.where(a_ref[...] == 0, jnp.float32(_NEG_BIG), logits)
        col = jax.lax.broadcasted_iota(jnp.int32, masked.shape, 1) + i * tile
        masked = jnp.where(col < vocab, masked, jnp.float32(_NEG_PAD))
        m_old = m_ref[:, 0:1]
        s_old = s_ref[:, 0:1]
        m_tile = jnp.max(masked, axis=1, keepdims=True)
        m_new = jnp.maximum(m_old, m_tile)
        s_new = (s_old * jnp.exp(m_old - m_new)
                 + jnp.sum(jnp.exp(masked - m_new), axis=1, keepdims=True))
        m_ref[:, 0:1] = m_new
        s_ref[:, 0:1] = s_new
        buf_ref[i] = masked

    @pl.when(i >= n_tiles)
    def _phase2():
        j = i - n_tiles
        lse = m_ref[:, 0:1] + jnp.log(s_ref[:, 0:1])
        o_ref[...] = buf_ref[j] - lse


def _build_call(batch, feat, vocab, tile):
    n = pl.cdiv(vocab, tile)
    body = functools.partial(_body, n, vocab, tile)
    grid = (2 * n,)
    in_specs = [
        pl.BlockSpec((batch, feat), lambda i: (0, 0)),
        pl.BlockSpec((batch, tile), lambda i: (0, jnp.minimum(i, n - 1))),
        pl.BlockSpec((tile, feat), lambda i: (jnp.minimum(i, n - 1), 0)),
        pl.BlockSpec((1, tile), lambda i: (0, jnp.minimum(i, n - 1))),
    ]
    out_spec = pl.BlockSpec((batch, tile), lambda i: (0, jnp.maximum(i - n, 0)))
    scratch = [
        pltpu.VMEM((n, batch, tile), jnp.float32),
        pltpu.VMEM((batch, 128), jnp.float32),
        pltpu.VMEM((batch, 128), jnp.float32),
    ]
    return pl.pallas_call(
        body,
        grid=grid,
        in_specs=in_specs,
        out_specs=out_spec,
        out_shape=jax.ShapeDtypeStruct((batch, vocab), jnp.float32),
        scratch_shapes=scratch,
        compiler_params=pltpu.CompilerParams(
            vmem_limit_bytes=100 * 1024 * 1024),
    )


def kernel(x, available_actions, W, b):
    batch, feat = x.shape
    vocab = W.shape[0]
    if available_actions.ndim == 1:
        available_actions = available_actions[None, :]
    available_actions = jnp.broadcast_to(available_actions, (batch, vocab))
    b2 = b.reshape(1, vocab)
    call = _build_call(batch, feat, vocab, _TILE)
    return call(x, available_actions, W, b2)
